# initial kernel scaffold (unmeasured)
import jax
import jax.numpy as jnp
from jax import lax
from jax.experimental import pallas as pl
from jax.experimental.pallas import tpu as pltpu

N_DEV = 4
TAPS = 4


def _compute_body(x_ref, k_ref, w_ref, o_ref):
    xv = x_ref[0]
    s, c = xv.shape
    conv = xv * k_ref[TAPS - 1]
    for t in range(TAPS - 1):
        m = TAPS - 1 - t
        shifted = jnp.concatenate(
            [jnp.zeros((m, c), jnp.float32), xv[: s - m]], axis=0
        )
        conv += shifted * k_ref[t]
    a = conv * jax.nn.sigmoid(conv)
    o_ref[0] = jnp.dot(a, w_ref[...], preferred_element_type=jnp.float32)


def _partial(x, k, Wp):
    B, S, C = x.shape
    Co = Wp.shape[1]
    return pl.pallas_call(
        _compute_body,
        grid=(B,),
        in_specs=[
            pl.BlockSpec((1, S, C), lambda b: (b, 0, 0)),
            pl.BlockSpec((TAPS, C), lambda b: (0, 0)),
            pl.BlockSpec((C, Co), lambda b: (0, 0)),
        ],
        out_specs=pl.BlockSpec((1, S, Co), lambda b: (b, 0, 0)),
        out_shape=jax.ShapeDtypeStruct((B, S, Co), jnp.float32),
    )(x, k, Wp)


def _ar_body(p_ref, o_ref, comm, acc, send_sems, recv_sems, cp_sem, st_sem):
    d = lax.axis_index("i")
    left = (d - 1) % N_DEV
    right = (d + 1) % N_DEV

    barrier = pltpu.get_barrier_semaphore()
    for nbr in (left, right):
        pl.semaphore_signal(
            barrier, inc=1, device_id=(nbr,),
            device_id_type=pl.DeviceIdType.MESH,
        )
    pl.semaphore_wait(barrier, 2)

    cp = pltpu.make_async_copy(p_ref.at[d], comm.at[0], cp_sem)
    cp.start()
    cp.wait()

    for h in range(N_DEV - 1):
        rs = (h + 1) % 2
        rdma = pltpu.make_async_remote_copy(
            src_ref=comm.at[h % 2],
            dst_ref=comm.at[rs],
            send_sem=send_sems.at[h],
            recv_sem=recv_sems.at[h],
            device_id=(right,),
            device_id_type=pl.DeviceIdType.MESH,
        )
        rdma.start()
        src_chunk = (d - h - 1) % N_DEV
        cp = pltpu.make_async_copy(p_ref.at[src_chunk], acc, cp_sem)
        cp.start()
        cp.wait()
        rdma.wait()
        comm[rs] = comm[rs] + acc[...]

    st = pltpu.make_async_copy(comm.at[1], o_ref.at[(d + 1) % N_DEV], st_sem)
    st.start()
    st.wait()

    for h in range(N_DEV - 1, 2 * (N_DEV - 1)):
        rs = (h + 1) % 2
        rdma = pltpu.make_async_remote_copy(
            src_ref=comm.at[h % 2],
            dst_ref=comm.at[rs],
            send_sem=send_sems.at[h],
            recv_sem=recv_sems.at[h],
            device_id=(right,),
            device_id_type=pl.DeviceIdType.MESH,
        )
        rdma.start()
        rdma.wait()
        chunk = (d - (h - (N_DEV - 1))) % N_DEV
        st = pltpu.make_async_copy(comm.at[rs], o_ref.at[chunk], st_sem)
        st.start()
        st.wait()


def _allreduce(p):
    B, S, Co = p.shape
    n_hops = 2 * (N_DEV - 1)
    return pl.pallas_call(
        _ar_body,
        in_specs=[pl.BlockSpec(memory_space=pltpu.ANY)],
        out_specs=pl.BlockSpec(memory_space=pltpu.ANY),
        out_shape=jax.ShapeDtypeStruct((B, S, Co), jnp.float32),
        scratch_shapes=[
            pltpu.VMEM((2, S, Co), jnp.float32),
            pltpu.VMEM((S, Co), jnp.float32),
            pltpu.SemaphoreType.DMA((n_hops,)),
            pltpu.SemaphoreType.DMA((n_hops,)),
            pltpu.SemaphoreType.DMA,
            pltpu.SemaphoreType.DMA,
        ],
        compiler_params=pltpu.CompilerParams(collective_id=0),
    )(p)


def kernel(x, k, Wp):
    return _allreduce(_partial(x, k, Wp))


# baseline (device time: 635553 ns/iter reference)
import jax
import jax.numpy as jnp
from jax import lax
from jax.experimental import pallas as pl
from jax.experimental.pallas import tpu as pltpu

N_DEV = 4
TAPS = 4


def _compute_body(x_ref, k_ref, w_ref, o_ref):
    xv = x_ref[0]
    s, c = xv.shape
    conv = xv * k_ref[TAPS - 1]
    for t in range(TAPS - 1):
        m = TAPS - 1 - t
        shifted = jnp.concatenate(
            [jnp.zeros((m, c), jnp.float32), xv[: s - m]], axis=0
        )
        conv += shifted * k_ref[t]
    a = conv * jax.nn.sigmoid(conv)
    o_ref[0] = jnp.dot(a, w_ref[...], preferred_element_type=jnp.float32)


def _partial(x, k, Wp):
    B, S, C = x.shape
    Co = Wp.shape[1]
    return pl.pallas_call(
        _compute_body,
        grid=(B,),
        in_specs=[
            pl.BlockSpec((1, S, C), lambda b: (b, 0, 0)),
            pl.BlockSpec((TAPS, C), lambda b: (0, 0)),
            pl.BlockSpec((C, Co), lambda b: (0, 0)),
        ],
        out_specs=pl.BlockSpec((1, S, Co), lambda b: (b, 0, 0)),
        out_shape=jax.ShapeDtypeStruct((B, S, Co), jnp.float32),
        compiler_params=pltpu.CompilerParams(
            vmem_limit_bytes=100 * 1024 * 1024
        ),
    )(x, k, Wp)


def _ar_body(p_ref, o_ref, comm, acc, send_sems, recv_sems, cp_sem, st_sem):
    d = lax.axis_index("i")
    left = (d - 1) % N_DEV
    right = (d + 1) % N_DEV

    barrier = pltpu.get_barrier_semaphore()
    for nbr in (left, right):
        pl.semaphore_signal(
            barrier, inc=1, device_id=(nbr,),
            device_id_type=pl.DeviceIdType.MESH,
        )
    pl.semaphore_wait(barrier, 2)

    cp = pltpu.make_async_copy(p_ref.at[d], comm.at[0], cp_sem)
    cp.start()
    cp.wait()

    for h in range(N_DEV - 1):
        rs = (h + 1) % 2
        rdma = pltpu.make_async_remote_copy(
            src_ref=comm.at[h % 2],
            dst_ref=comm.at[rs],
            send_sem=send_sems.at[h],
            recv_sem=recv_sems.at[h],
            device_id=(right,),
            device_id_type=pl.DeviceIdType.MESH,
        )
        rdma.start()
        src_chunk = (d - h - 1) % N_DEV
        cp = pltpu.make_async_copy(p_ref.at[src_chunk], acc, cp_sem)
        cp.start()
        cp.wait()
        rdma.wait()
        comm[rs] = comm[rs] + acc[...]

    st = pltpu.make_async_copy(comm.at[1], o_ref.at[(d + 1) % N_DEV], st_sem)
    st.start()
    st.wait()

    for h in range(N_DEV - 1, 2 * (N_DEV - 1)):
        rs = (h + 1) % 2
        rdma = pltpu.make_async_remote_copy(
            src_ref=comm.at[h % 2],
            dst_ref=comm.at[rs],
            send_sem=send_sems.at[h],
            recv_sem=recv_sems.at[h],
            device_id=(right,),
            device_id_type=pl.DeviceIdType.MESH,
        )
        rdma.start()
        rdma.wait()
        chunk = (d - (h - (N_DEV - 1))) % N_DEV
        st = pltpu.make_async_copy(comm.at[rs], o_ref.at[chunk], st_sem)
        st.start()
        st.wait()


def _allreduce(p):
    B, S, Co = p.shape
    n_hops = 2 * (N_DEV - 1)
    return pl.pallas_call(
        _ar_body,
        in_specs=[pl.BlockSpec(memory_space=pl.ANY)],
        out_specs=pl.BlockSpec(memory_space=pl.ANY),
        out_shape=jax.ShapeDtypeStruct((B, S, Co), jnp.float32),
        scratch_shapes=[
            pltpu.VMEM((2, S, Co), jnp.float32),
            pltpu.VMEM((S, Co), jnp.float32),
            pltpu.SemaphoreType.DMA((n_hops,)),
            pltpu.SemaphoreType.DMA((n_hops,)),
            pltpu.SemaphoreType.DMA,
            pltpu.SemaphoreType.DMA,
        ],
        compiler_params=pltpu.CompilerParams(
            collective_id=0, vmem_limit_bytes=100 * 1024 * 1024
        ),
    )(p)


def kernel(x, k, Wp):
    return _allreduce(_partial(x, k, Wp))


# device time: 356942 ns/iter; 1.7805x vs baseline; 1.7805x over previous
import jax
import jax.numpy as jnp
from jax import lax
from jax.experimental import pallas as pl
from jax.experimental.pallas import tpu as pltpu

N_DEV = 4
TAPS = 4


def _compute_body(x_ref, k_ref, w_ref, o_ref):
    xv = x_ref[0]
    s, c = xv.shape
    conv = xv * k_ref[TAPS - 1]
    for t in range(TAPS - 1):
        m = TAPS - 1 - t
        shifted = jnp.concatenate(
            [jnp.zeros((m, c), jnp.float32), xv[: s - m]], axis=0
        )
        conv += shifted * k_ref[t]
    a = conv * jax.nn.sigmoid(conv)
    o_ref[0] = jnp.dot(a, w_ref[...], preferred_element_type=jnp.float32)


def _partial(x, k, Wp):
    B, S, C = x.shape
    Co = Wp.shape[1]
    return pl.pallas_call(
        _compute_body,
        grid=(B,),
        in_specs=[
            pl.BlockSpec((1, S, C), lambda b: (b, 0, 0)),
            pl.BlockSpec((TAPS, C), lambda b: (0, 0)),
            pl.BlockSpec((C, Co), lambda b: (0, 0)),
        ],
        out_specs=pl.BlockSpec((1, S, Co), lambda b: (b, 0, 0)),
        out_shape=jax.ShapeDtypeStruct((B, S, Co), jnp.float32),
        compiler_params=pltpu.CompilerParams(
            vmem_limit_bytes=100 * 1024 * 1024
        ),
    )(x, k, Wp)


def _ar_body(
    p_ref, o_ref, comm_cw, comm_ccw, acc_cw, acc_ccw,
    send_cw, recv_cw, send_ccw, recv_ccw, cp_sem_cw, cp_sem_ccw, st_sems,
):
    d = lax.axis_index("i")
    left = (d - 1) % N_DEV
    right = (d + 1) % N_DEV
    S = p_ref.shape[1]
    H = S // 2

    barrier = pltpu.get_barrier_semaphore()
    for nbr in (left, right):
        pl.semaphore_signal(
            barrier, inc=1, device_id=(nbr,),
            device_id_type=pl.DeviceIdType.MESH,
        )
    pl.semaphore_wait(barrier, 2)

    cp0 = pltpu.make_async_copy(
        p_ref.at[d, pl.ds(0, H)], comm_cw.at[0], cp_sem_cw)
    cp1 = pltpu.make_async_copy(
        p_ref.at[d, pl.ds(H, H)], comm_ccw.at[0], cp_sem_ccw)
    cp0.start()
    cp1.start()
    cp0.wait()
    cp1.wait()

    for h in range(N_DEV - 1):
        rs = (h + 1) % 2
        rdma_cw = pltpu.make_async_remote_copy(
            src_ref=comm_cw.at[h % 2],
            dst_ref=comm_cw.at[rs],
            send_sem=send_cw.at[h],
            recv_sem=recv_cw.at[h],
            device_id=(right,),
            device_id_type=pl.DeviceIdType.MESH,
        )
        rdma_ccw = pltpu.make_async_remote_copy(
            src_ref=comm_ccw.at[h % 2],
            dst_ref=comm_ccw.at[rs],
            send_sem=send_ccw.at[h],
            recv_sem=recv_ccw.at[h],
            device_id=(left,),
            device_id_type=pl.DeviceIdType.MESH,
        )
        rdma_cw.start()
        rdma_ccw.start()
        cr_cw = (d - h - 1) % N_DEV
        cr_ccw = (d + h + 1) % N_DEV
        cp0 = pltpu.make_async_copy(
            p_ref.at[cr_cw, pl.ds(0, H)], acc_cw, cp_sem_cw)
        cp1 = pltpu.make_async_copy(
            p_ref.at[cr_ccw, pl.ds(H, H)], acc_ccw, cp_sem_ccw)
        cp0.start()
        cp1.start()
        cp0.wait()
        cp1.wait()
        rdma_cw.wait()
        rdma_ccw.wait()
        comm_cw[rs] = comm_cw[rs] + acc_cw[...]
        comm_ccw[rs] = comm_ccw[rs] + acc_ccw[...]

    stores = []
    st = pltpu.make_async_copy(
        comm_cw.at[1], o_ref.at[(d + 1) % N_DEV, pl.ds(0, H)], st_sems.at[0])
    st.start()
    stores.append(st)
    st = pltpu.make_async_copy(
        comm_ccw.at[1], o_ref.at[(d - 1) % N_DEV, pl.ds(H, H)], st_sems.at[1])
    st.start()
    stores.append(st)

    for h in range(N_DEV - 1, 2 * (N_DEV - 1)):
        rs = (h + 1) % 2
        rdma_cw = pltpu.make_async_remote_copy(
            src_ref=comm_cw.at[h % 2],
            dst_ref=comm_cw.at[rs],
            send_sem=send_cw.at[h],
            recv_sem=recv_cw.at[h],
            device_id=(right,),
            device_id_type=pl.DeviceIdType.MESH,
        )
        rdma_ccw = pltpu.make_async_remote_copy(
            src_ref=comm_ccw.at[h % 2],
            dst_ref=comm_ccw.at[rs],
            send_sem=send_ccw.at[h],
            recv_sem=recv_ccw.at[h],
            device_id=(left,),
            device_id_type=pl.DeviceIdType.MESH,
        )
        rdma_cw.start()
        rdma_ccw.start()
        rdma_cw.wait()
        rdma_ccw.wait()
        j = h - (N_DEV - 1)
        c_cw = (d - j) % N_DEV
        c_ccw = (d + j) % N_DEV
        st = pltpu.make_async_copy(
            comm_cw.at[rs], o_ref.at[c_cw, pl.ds(0, H)],
            st_sems.at[2 * j + 2])
        st.start()
        stores.append(st)
        st = pltpu.make_async_copy(
            comm_ccw.at[rs], o_ref.at[c_ccw, pl.ds(H, H)],
            st_sems.at[2 * j + 3])
        st.start()
        stores.append(st)

    for st in stores:
        st.wait()


def _allreduce(p):
    B, S, Co = p.shape
    H = S // 2
    n_hops = 2 * (N_DEV - 1)
    return pl.pallas_call(
        _ar_body,
        in_specs=[pl.BlockSpec(memory_space=pl.ANY)],
        out_specs=pl.BlockSpec(memory_space=pl.ANY),
        out_shape=jax.ShapeDtypeStruct((B, S, Co), jnp.float32),
        scratch_shapes=[
            pltpu.VMEM((2, H, Co), jnp.float32),
            pltpu.VMEM((2, H, Co), jnp.float32),
            pltpu.VMEM((H, Co), jnp.float32),
            pltpu.VMEM((H, Co), jnp.float32),
            pltpu.SemaphoreType.DMA((n_hops,)),
            pltpu.SemaphoreType.DMA((n_hops,)),
            pltpu.SemaphoreType.DMA((n_hops,)),
            pltpu.SemaphoreType.DMA((n_hops,)),
            pltpu.SemaphoreType.DMA,
            pltpu.SemaphoreType.DMA,
            pltpu.SemaphoreType.DMA((8,)),
        ],
        compiler_params=pltpu.CompilerParams(
            collective_id=0, vmem_limit_bytes=100 * 1024 * 1024
        ),
    )(p)


def kernel(x, k, Wp):
    return _allreduce(_partial(x, k, Wp))


# device time: 223429 ns/iter; 2.8445x vs baseline; 1.5976x over previous
import jax
import jax.numpy as jnp
from jax import lax
from jax.experimental import pallas as pl
from jax.experimental.pallas import tpu as pltpu

N_DEV = 4
TAPS = 4


def _compute_body(x_ref, k_ref, w_ref, o_ref):
    xv = x_ref[0]
    s, c = xv.shape
    conv = xv * k_ref[TAPS - 1]
    for t in range(TAPS - 1):
        m = TAPS - 1 - t
        shifted = jnp.concatenate(
            [jnp.zeros((m, c), jnp.float32), xv[: s - m]], axis=0
        )
        conv += shifted * k_ref[t]
    a = conv * jax.nn.sigmoid(conv)
    o_ref[0] = jnp.dot(a, w_ref[...], preferred_element_type=jnp.float32)


def _partial(x, k, Wp):
    B, S, C = x.shape
    Co = Wp.shape[1]
    return pl.pallas_call(
        _compute_body,
        grid=(B,),
        in_specs=[
            pl.BlockSpec((1, S, C), lambda b: (b, 0, 0)),
            pl.BlockSpec((TAPS, C), lambda b: (0, 0)),
            pl.BlockSpec((C, Co), lambda b: (0, 0)),
        ],
        out_specs=pl.BlockSpec((1, S, Co), lambda b: (b, 0, 0)),
        out_shape=jax.ShapeDtypeStruct((B, S, Co), jnp.float32),
        compiler_params=pltpu.CompilerParams(
            vmem_limit_bytes=100 * 1024 * 1024
        ),
    )(x, k, Wp)


def _ar_body(
    p_ref, o_ref, comm_cw, comm_ccw, acc_cw, acc_ccw, stage_cw, stage_ccw,
    send_cw, recv_cw, send_ccw, recv_ccw, cp_sem_cw, cp_sem_ccw,
    st_cw, st_ccw,
):
    d = lax.axis_index("i")
    left = (d - 1) % N_DEV
    right = (d + 1) % N_DEV
    S = p_ref.shape[1]
    H = S // 2
    f32 = jnp.float32
    bf16 = jnp.bfloat16

    barrier = pltpu.get_barrier_semaphore()
    for nbr in (left, right):
        pl.semaphore_signal(
            barrier, inc=1, device_id=(nbr,),
            device_id_type=pl.DeviceIdType.MESH,
        )
    pl.semaphore_wait(barrier, 2)

    cp0 = pltpu.make_async_copy(
        p_ref.at[d, pl.ds(0, H)], acc_cw, cp_sem_cw)
    cp1 = pltpu.make_async_copy(
        p_ref.at[d, pl.ds(H, H)], acc_ccw, cp_sem_ccw)
    cp0.start()
    cp1.start()
    cp0.wait()
    cp1.wait()
    comm_cw[0] = acc_cw[...].astype(bf16)
    comm_ccw[0] = acc_ccw[...].astype(bf16)

    for h in range(N_DEV - 1):
        rs = (h + 1) % 2
        rdma_cw = pltpu.make_async_remote_copy(
            src_ref=comm_cw.at[h % 2],
            dst_ref=comm_cw.at[rs],
            send_sem=send_cw.at[h],
            recv_sem=recv_cw.at[h],
            device_id=(right,),
            device_id_type=pl.DeviceIdType.MESH,
        )
        rdma_ccw = pltpu.make_async_remote_copy(
            src_ref=comm_ccw.at[h % 2],
            dst_ref=comm_ccw.at[rs],
            send_sem=send_ccw.at[h],
            recv_sem=recv_ccw.at[h],
            device_id=(left,),
            device_id_type=pl.DeviceIdType.MESH,
        )
        rdma_cw.start()
        rdma_ccw.start()
        cr_cw = (d - h - 1) % N_DEV
        cr_ccw = (d + h + 1) % N_DEV
        cp0 = pltpu.make_async_copy(
            p_ref.at[cr_cw, pl.ds(0, H)], acc_cw, cp_sem_cw)
        cp1 = pltpu.make_async_copy(
            p_ref.at[cr_ccw, pl.ds(H, H)], acc_ccw, cp_sem_ccw)
        cp0.start()
        cp1.start()
        cp0.wait()
        cp1.wait()
        rdma_cw.wait()
        rdma_ccw.wait()
        comm_cw[rs] = (comm_cw[rs].astype(f32) + acc_cw[...]).astype(bf16)
        comm_ccw[rs] = (comm_ccw[rs].astype(f32) + acc_ccw[...]).astype(bf16)

    stores_cw = []
    stores_ccw = []
    stage_cw[0] = comm_cw[1].astype(f32)
    st = pltpu.make_async_copy(
        stage_cw.at[0], o_ref.at[(d + 1) % N_DEV, pl.ds(0, H)], st_cw.at[0])
    st.start()
    stores_cw.append(st)
    stage_ccw[0] = comm_ccw[1].astype(f32)
    st = pltpu.make_async_copy(
        stage_ccw.at[0], o_ref.at[(d - 1) % N_DEV, pl.ds(H, H)], st_ccw.at[0])
    st.start()
    stores_ccw.append(st)

    for h in range(N_DEV - 1, 2 * (N_DEV - 1)):
        rs = (h + 1) % 2
        rdma_cw = pltpu.make_async_remote_copy(
            src_ref=comm_cw.at[h % 2],
            dst_ref=comm_cw.at[rs],
            send_sem=send_cw.at[h],
            recv_sem=recv_cw.at[h],
            device_id=(right,),
            device_id_type=pl.DeviceIdType.MESH,
        )
        rdma_ccw = pltpu.make_async_remote_copy(
            src_ref=comm_ccw.at[h % 2],
            dst_ref=comm_ccw.at[rs],
            send_sem=send_ccw.at[h],
            recv_sem=recv_ccw.at[h],
            device_id=(left,),
            device_id_type=pl.DeviceIdType.MESH,
        )
        rdma_cw.start()
        rdma_ccw.start()
        rdma_cw.wait()
        rdma_ccw.wait()
        j = h - (N_DEV - 1)
        c_cw = (d - j) % N_DEV
        c_ccw = (d + j) % N_DEV
        s = j + 1
        slot = s % 2
        if s >= 2:
            stores_cw[s - 2].wait()
            stores_ccw[s - 2].wait()
        stage_cw[slot] = comm_cw[rs].astype(f32)
        st = pltpu.make_async_copy(
            stage_cw.at[slot], o_ref.at[c_cw, pl.ds(0, H)], st_cw.at[s])
        st.start()
        stores_cw.append(st)
        stage_ccw[slot] = comm_ccw[rs].astype(f32)
        st = pltpu.make_async_copy(
            stage_ccw.at[slot], o_ref.at[c_ccw, pl.ds(H, H)], st_ccw.at[s])
        st.start()
        stores_ccw.append(st)

    for st in stores_cw[-2:] + stores_ccw[-2:]:
        st.wait()


def _allreduce(p):
    B, S, Co = p.shape
    H = S // 2
    n_hops = 2 * (N_DEV - 1)
    return pl.pallas_call(
        _ar_body,
        in_specs=[pl.BlockSpec(memory_space=pl.ANY)],
        out_specs=pl.BlockSpec(memory_space=pl.ANY),
        out_shape=jax.ShapeDtypeStruct((B, S, Co), jnp.float32),
        scratch_shapes=[
            pltpu.VMEM((2, H, Co), jnp.bfloat16),
            pltpu.VMEM((2, H, Co), jnp.bfloat16),
            pltpu.VMEM((H, Co), jnp.float32),
            pltpu.VMEM((H, Co), jnp.float32),
            pltpu.VMEM((2, H, Co), jnp.float32),
            pltpu.VMEM((2, H, Co), jnp.float32),
            pltpu.SemaphoreType.DMA((n_hops,)),
            pltpu.SemaphoreType.DMA((n_hops,)),
            pltpu.SemaphoreType.DMA((n_hops,)),
            pltpu.SemaphoreType.DMA((n_hops,)),
            pltpu.SemaphoreType.DMA,
            pltpu.SemaphoreType.DMA,
            pltpu.SemaphoreType.DMA((4,)),
            pltpu.SemaphoreType.DMA((4,)),
        ],
        compiler_params=pltpu.CompilerParams(
            collective_id=0, vmem_limit_bytes=100 * 1024 * 1024
        ),
    )(p)


def kernel(x, k, Wp):
    return _allreduce(_partial(x, k, Wp))


# device time: 191032 ns/iter; 3.3269x vs baseline; 1.1696x over previous
import jax
import jax.numpy as jnp
from jax import lax
from jax.experimental import pallas as pl
from jax.experimental.pallas import tpu as pltpu

N_DEV = 4
TAPS = 4
HALO = 8


def _silu(v):
    return v * jax.nn.sigmoid(v)


def _fused_body(
    x_ref, k_ref, w_ref, o_ref,
    comm_cw, comm_ccw, xs, xb, stage_cw, stage_ccw,
    send_cw, recv_cw, send_ccw, recv_ccw, x_sem_t, x_sem_b, st_cw, st_ccw,
):
    d = lax.axis_index("i")
    left = (d - 1) % N_DEV
    right = (d + 1) % N_DEV
    B, S, C = x_ref.shape
    H = S // 2
    f32 = jnp.float32
    bf16 = jnp.bfloat16
    w_bf = w_ref[...].astype(bf16)

    def load_top(b):
        cp = pltpu.make_async_copy(x_ref.at[b, pl.ds(0, H)], xs, x_sem_t)
        cp.start()
        return cp

    def load_bot(b):
        cp = pltpu.make_async_copy(
            x_ref.at[b, pl.ds(H - HALO, H + HALO)], xb, x_sem_b)
        cp.start()
        return cp

    def product_top():
        xv = xs[...]
        conv = xv * k_ref[TAPS - 1]
        for t in range(TAPS - 1):
            m = TAPS - 1 - t
            conv += jnp.concatenate(
                [jnp.zeros((m, C), f32), xv[: H - m]], axis=0
            ) * k_ref[t]
        return jnp.dot(
            _silu(conv).astype(bf16), w_bf, preferred_element_type=f32
        )

    def product_bot():
        xv = xb[...]
        conv = xv[HALO: HALO + H] * k_ref[TAPS - 1]
        for t in range(TAPS - 1):
            m = TAPS - 1 - t
            conv += xv[HALO - m: HALO - m + H] * k_ref[t]
        return jnp.dot(
            _silu(conv).astype(bf16), w_bf, preferred_element_type=f32
        )

    cpT = load_top(d)
    cpB = load_bot(d)
    cpT.wait()
    comm_cw[0] = product_top().astype(bf16)
    cpB.wait()
    comm_ccw[0] = product_bot().astype(bf16)

    barrier = pltpu.get_barrier_semaphore()
    for nbr in (left, right):
        pl.semaphore_signal(
            barrier, inc=1, device_id=(nbr,),
            device_id_type=pl.DeviceIdType.MESH,
        )
    pl.semaphore_wait(barrier, 2)

    for h in range(N_DEV - 1):
        rs = (h + 1) % 2
        rdma_cw = pltpu.make_async_remote_copy(
            src_ref=comm_cw.at[h % 2],
            dst_ref=comm_cw.at[rs],
            send_sem=send_cw.at[h],
            recv_sem=recv_cw.at[h],
            device_id=(right,),
            device_id_type=pl.DeviceIdType.MESH,
        )
        rdma_ccw = pltpu.make_async_remote_copy(
            src_ref=comm_ccw.at[h % 2],
            dst_ref=comm_ccw.at[rs],
            send_sem=send_ccw.at[h],
            recv_sem=recv_ccw.at[h],
            device_id=(left,),
            device_id_type=pl.DeviceIdType.MESH,
        )
        rdma_cw.start()
        rdma_ccw.start()
        cpT = load_top((d - h - 1) % N_DEV)
        cpB = load_bot((d + h + 1) % N_DEV)
        cpT.wait()
        add_cw = product_top()
        cpB.wait()
        add_ccw = product_bot()
        rdma_cw.wait()
        rdma_ccw.wait()
        comm_cw[rs] = (comm_cw[rs].astype(f32) + add_cw).astype(bf16)
        comm_ccw[rs] = (comm_ccw[rs].astype(f32) + add_ccw).astype(bf16)

    stores_cw = []
    stores_ccw = []
    stage_cw[0] = comm_cw[1].astype(f32)
    st = pltpu.make_async_copy(
        stage_cw.at[0], o_ref.at[(d + 1) % N_DEV, pl.ds(0, H)], st_cw.at[0])
    st.start()
    stores_cw.append(st)
    stage_ccw[0] = comm_ccw[1].astype(f32)
    st = pltpu.make_async_copy(
        stage_ccw.at[0], o_ref.at[(d - 1) % N_DEV, pl.ds(H, H)], st_ccw.at[0])
    st.start()
    stores_ccw.append(st)

    for h in range(N_DEV - 1, 2 * (N_DEV - 1)):
        rs = (h + 1) % 2
        rdma_cw = pltpu.make_async_remote_copy(
            src_ref=comm_cw.at[h % 2],
            dst_ref=comm_cw.at[rs],
            send_sem=send_cw.at[h],
            recv_sem=recv_cw.at[h],
            device_id=(right,),
            device_id_type=pl.DeviceIdType.MESH,
        )
        rdma_ccw = pltpu.make_async_remote_copy(
            src_ref=comm_ccw.at[h % 2],
            dst_ref=comm_ccw.at[rs],
            send_sem=send_ccw.at[h],
            recv_sem=recv_ccw.at[h],
            device_id=(left,),
            device_id_type=pl.DeviceIdType.MESH,
        )
        rdma_cw.start()
        rdma_ccw.start()
        rdma_cw.wait()
        rdma_ccw.wait()
        j = h - (N_DEV - 1)
        c_cw = (d - j) % N_DEV
        c_ccw = (d + j) % N_DEV
        s = j + 1
        slot = s % 2
        if s >= 2:
            stores_cw[s - 2].wait()
            stores_ccw[s - 2].wait()
        stage_cw[slot] = comm_cw[rs].astype(f32)
        st = pltpu.make_async_copy(
            stage_cw.at[slot], o_ref.at[c_cw, pl.ds(0, H)], st_cw.at[s])
        st.start()
        stores_cw.append(st)
        stage_ccw[slot] = comm_ccw[rs].astype(f32)
        st = pltpu.make_async_copy(
            stage_ccw.at[slot], o_ref.at[c_ccw, pl.ds(H, H)], st_ccw.at[s])
        st.start()
        stores_ccw.append(st)

    for st in stores_cw[-2:] + stores_ccw[-2:]:
        st.wait()


def kernel(x, k, Wp):
    B, S, C = x.shape
    Co = Wp.shape[1]
    H = S // 2
    n_hops = 2 * (N_DEV - 1)
    return pl.pallas_call(
        _fused_body,
        in_specs=[
            pl.BlockSpec(memory_space=pl.ANY),
            pl.BlockSpec(memory_space=pltpu.VMEM),
            pl.BlockSpec(memory_space=pltpu.VMEM),
        ],
        out_specs=pl.BlockSpec(memory_space=pl.ANY),
        out_shape=jax.ShapeDtypeStruct((B, S, Co), jnp.float32),
        scratch_shapes=[
            pltpu.VMEM((2, H, Co), jnp.bfloat16),
            pltpu.VMEM((2, H, Co), jnp.bfloat16),
            pltpu.VMEM((H, C), jnp.float32),
            pltpu.VMEM((H + HALO, C), jnp.float32),
            pltpu.VMEM((2, H, Co), jnp.float32),
            pltpu.VMEM((2, H, Co), jnp.float32),
            pltpu.SemaphoreType.DMA((n_hops,)),
            pltpu.SemaphoreType.DMA((n_hops,)),
            pltpu.SemaphoreType.DMA((n_hops,)),
            pltpu.SemaphoreType.DMA((n_hops,)),
            pltpu.SemaphoreType.DMA,
            pltpu.SemaphoreType.DMA,
            pltpu.SemaphoreType.DMA((4,)),
            pltpu.SemaphoreType.DMA((4,)),
        ],
        compiler_params=pltpu.CompilerParams(
            collective_id=0, vmem_limit_bytes=110 * 1024 * 1024
        ),
    )(x, k, Wp)
